# baseline (speedup 1.0000x reference)
import functools

import jax
import jax.numpy as jnp
from jax import lax
from jax.experimental import pallas as pl
from jax.experimental.pallas import tpu as pltpu
from jax.experimental.pallas import tpu_sc as plsc

D_MODEL = 768
SEQ = 2048
BATCH = 4
NC = 2
NS = 16
L = 16
NW = NC * NS
POS_PER_W = SEQ // NW
CHUNK = 16
POS_CHUNKS = POS_PER_W // CHUNK
N_CHUNKS = BATCH * POS_CHUNKS
VECS_PER_ROW = D_MODEL // L
NBUF = 4
N_POS_BUF = 2


def _sc_embed(idx_hbm, table_hbm, pos_hbm):
    mesh = plsc.VectorSubcoreMesh(core_axis_name="c", subcore_axis_name="s")

    @functools.partial(
        pl.kernel,
        out_type=(
            jax.ShapeDtypeStruct((BATCH * SEQ, D_MODEL), jnp.float32),
            jax.ShapeDtypeStruct((BATCH * SEQ, D_MODEL), jnp.float32),
        ),
        mesh=mesh,
        scratch_types=[
            pltpu.VMEM((BATCH, POS_PER_W), jnp.int32),
            pltpu.VMEM((NBUF, CHUNK, D_MODEL), jnp.float32),
            pltpu.VMEM((NBUF, CHUNK, D_MODEL), jnp.float32),
            pltpu.SemaphoreType.DMA,
            pltpu.SemaphoreType.DMA((NBUF,)),
            pltpu.SemaphoreType.DMA((NBUF,)),
            pltpu.SemaphoreType.DMA((NBUF,)),
            pltpu.SemaphoreType.DMA((NBUF,)),
        ],
    )
    def k(idx_ref, table_ref, pos_ref, emb_out, wo_out,
          idx_v, rows_v, emb_v, isem, gsem, wsem, csem, esem):
        wid = lax.axis_index("s") * NC + lax.axis_index("c")
        pos_base = wid * POS_PER_W

        def gather(c, b, p):
            return pltpu.async_copy(
                table_ref.at[idx_v.at[b, pl.ds(c * CHUNK, CHUNK)]],
                rows_v.at[p], gsem.at[p])

        def pos_init(c, p):
            return pltpu.async_copy(
                pos_ref.at[pl.ds(pos_base + c * CHUNK, CHUNK)], emb_v.at[p],
                csem.at[p])

        def wo_write(c, b, p):
            return pltpu.async_copy(
                rows_v.at[p],
                wo_out.at[pl.ds(b * SEQ + pos_base + c * CHUNK, CHUNK)],
                wsem.at[p])

        def emb_write(c, b, p):
            return pltpu.async_copy(
                emb_v.at[p],
                emb_out.at[pl.ds(b * SEQ + pos_base + c * CHUNK, CHUNK)],
                esem.at[p])

        def wait_w(p):
            pltpu.make_async_copy(
                rows_v.at[p], wo_out.at[pl.ds(0, CHUNK)], wsem.at[p]).wait()

        def wait_e(p):
            pltpu.make_async_copy(
                emb_v.at[p], emb_out.at[pl.ds(0, CHUNK)], esem.at[p]).wait()

        idx_cps = [
            pltpu.async_copy(
                idx_ref.at[b, pl.ds(pos_base, POS_PER_W)], idx_v.at[b], isem)
            for b in range(BATCH)
        ]
        for cp in idx_cps:
            cp.wait()
        gather(0, 0, 0)
        gather(0, 1, 1)
        pos_init(0, 0)
        pos_init(0, 1)

        def chunk(i, carry):
            c = lax.div(i, BATCH)
            b = lax.rem(i, BATCH)
            p = lax.rem(i, NBUF)

            @pl.when(i + 2 < N_CHUNKS)
            def _():
                i2 = i + 2
                p2 = lax.rem(i2, NBUF)

                @pl.when(i >= 2)
                def _():
                    wait_w(p2)
                    wait_e(p2)

                c2 = lax.div(i2, BATCH)
                b2 = lax.rem(i2, BATCH)
                gather(c2, b2, p2)
                pos_init(c2, p2)

            pltpu.make_async_copy(
                table_ref.at[idx_v.at[b, pl.ds(c * CHUNK, CHUNK)]],
                rows_v.at[p], gsem.at[p]).wait()
            wo_write(c, b, p)
            pltpu.make_async_copy(
                pos_ref.at[pl.ds(pos_base, CHUNK)], emb_v.at[p],
                csem.at[p]).wait()

            def body(r, carry2):
                for kk in range(VECS_PER_ROW):
                    sl = pl.ds(kk * L, L)
                    plsc.addupdate(emb_v.at[p, r, sl], rows_v[p, r, sl])
                return carry2

            lax.fori_loop(0, CHUNK, body, 0)
            emb_write(c, b, p)
            return carry

        lax.fori_loop(0, N_CHUNKS, chunk, 0)

        for p in range(NBUF):
            wait_w(p)
            wait_e(p)

    return k(idx_hbm, table_hbm, pos_hbm)


def kernel(inputs, token_embeddings, position_embeddings):
    idx = inputs.astype(jnp.int32)
    emb_flat, wo_flat = _sc_embed(idx, token_embeddings, position_embeddings)
    emb = emb_flat.reshape(BATCH, SEQ, D_MODEL)
    wo = wo_flat.reshape(BATCH, SEQ, D_MODEL)
    return (emb, None, wo)


# speedup vs baseline: 1.1271x; 1.1271x over previous
import functools

import jax
import jax.numpy as jnp
from jax import lax
from jax.experimental import pallas as pl
from jax.experimental.pallas import tpu as pltpu
from jax.experimental.pallas import tpu_sc as plsc

D_MODEL = 768
SEQ = 2048
BATCH = 4
NC = 2
NS = 16
L = 16
NW = NC * NS
POS_PER_W = SEQ // NW
CHUNK = 16
POS_CHUNKS = POS_PER_W // CHUNK
N_CHUNKS = BATCH * POS_CHUNKS
VECS_PER_ROW = D_MODEL // L
NBUF = 4
N_POS_BUF = 2


def _sc_embed(idx_hbm, table_hbm, pos_hbm):
    mesh = plsc.VectorSubcoreMesh(core_axis_name="c", subcore_axis_name="s")

    @functools.partial(
        pl.kernel,
        out_type=(
            jax.ShapeDtypeStruct((BATCH * SEQ, D_MODEL), jnp.float32),
            jax.ShapeDtypeStruct((BATCH * SEQ, D_MODEL), jnp.float32),
        ),
        mesh=mesh,
        scratch_types=[
            pltpu.VMEM((BATCH, POS_PER_W), jnp.int32),
            pltpu.VMEM_SHARED(
                (NS, N_POS_BUF, CHUNK, D_MODEL), jnp.float32),
            pltpu.VMEM((NBUF, CHUNK, D_MODEL), jnp.float32),
            pltpu.VMEM((NBUF, CHUNK, D_MODEL), jnp.float32),
            pltpu.SemaphoreType.DMA((BATCH,)),
            pltpu.SemaphoreType.DMA,
            pltpu.SemaphoreType.DMA((NBUF,)),
            pltpu.SemaphoreType.DMA((NBUF,)),
            pltpu.SemaphoreType.DMA((NBUF,)),
            pltpu.SemaphoreType.DMA((NBUF,)),
        ],
    )
    def k(idx_ref, table_ref, pos_ref, emb_out, wo_out,
          idx_v, pos_v, rows_v, emb_v, isem, psem, gsem, wsem, csem, esem):
        sid = lax.axis_index("s")
        wid = sid * NC + lax.axis_index("c")
        pos_base = wid * POS_PER_W

        def pos_load(c):
            return pltpu.async_copy(
                pos_ref.at[pl.ds(pos_base + c * CHUNK, CHUNK)],
                pos_v.at[sid, lax.rem(c, N_POS_BUF)], psem)

        def gather(c, b, p):
            return pltpu.async_copy(
                table_ref.at[idx_v.at[b, pl.ds(c * CHUNK, CHUNK)]],
                rows_v.at[p], gsem.at[p])

        def pos_init(c, p):
            return pltpu.async_copy(
                pos_v.at[sid, lax.rem(c, N_POS_BUF)], emb_v.at[p],
                csem.at[p])

        def wo_write(c, b, p):
            return pltpu.async_copy(
                rows_v.at[p],
                wo_out.at[pl.ds(b * SEQ + pos_base + c * CHUNK, CHUNK)],
                wsem.at[p])

        def emb_write(c, b, p):
            return pltpu.async_copy(
                emb_v.at[p],
                emb_out.at[pl.ds(b * SEQ + pos_base + c * CHUNK, CHUNK)],
                esem.at[p])

        def wait_w(p):
            pltpu.make_async_copy(
                rows_v.at[p], wo_out.at[pl.ds(0, CHUNK)], wsem.at[p]).wait()

        def wait_e(p):
            pltpu.make_async_copy(
                emb_v.at[p], emb_out.at[pl.ds(0, CHUNK)], esem.at[p]).wait()

        pl0 = pos_load(0)
        idx_cps = [
            pltpu.async_copy(
                idx_ref.at[b, pl.ds(pos_base, POS_PER_W)], idx_v.at[b],
                isem.at[b])
            for b in range(BATCH)
        ]
        idx_cps[0].wait()
        gather(0, 0, 0)
        idx_cps[1].wait()
        gather(0, 1, 1)
        pl0.wait()
        pos_init(0, 0)
        pos_init(0, 1)

        def chunk(i, carry):
            c = lax.div(i, BATCH)
            b = lax.rem(i, BATCH)
            p = lax.rem(i, NBUF)

            @pl.when(i + 2 < N_CHUNKS)
            def _():
                i2 = i + 2
                p2 = lax.rem(i2, NBUF)

                @pl.when(i >= 2)
                def _():
                    wait_w(p2)
                    wait_e(p2)

                c2 = lax.div(i2, BATCH)
                b2 = lax.rem(i2, BATCH)

                @pl.when(i < 2)
                def _():
                    pltpu.make_async_copy(
                        idx_ref.at[b2, pl.ds(pos_base, POS_PER_W)],
                        idx_v.at[b2], isem.at[b2]).wait()

                @pl.when(b2 == 0)
                def _():
                    pltpu.make_async_copy(
                        pos_ref.at[pl.ds(pos_base, CHUNK)],
                        pos_v.at[sid, lax.rem(c2, N_POS_BUF)], psem).wait()

                gather(c2, b2, p2)
                pos_init(c2, p2)

            @pl.when(jnp.logical_and(b == 0, c + 1 < POS_CHUNKS))
            def _():
                pos_load(c + 1)

            pltpu.make_async_copy(
                table_ref.at[idx_v.at[b, pl.ds(c * CHUNK, CHUNK)]],
                rows_v.at[p], gsem.at[p]).wait()
            wo_write(c, b, p)
            pltpu.make_async_copy(
                pos_v.at[sid, lax.rem(c, N_POS_BUF)], emb_v.at[p],
                csem.at[p]).wait()

            def body(r, carry2):
                for kk in range(VECS_PER_ROW):
                    sl = pl.ds(kk * L, L)
                    plsc.addupdate(emb_v.at[p, r, sl], rows_v[p, r, sl])
                return carry2

            lax.fori_loop(0, CHUNK, body, 0)
            emb_write(c, b, p)
            return carry

        lax.fori_loop(0, N_CHUNKS, chunk, 0)

        for p in range(NBUF):
            wait_w(p)
            wait_e(p)

    return k(idx_hbm, table_hbm, pos_hbm)


def kernel(inputs, token_embeddings, position_embeddings):
    idx = inputs.astype(jnp.int32)
    emb_flat, wo_flat = _sc_embed(idx, token_embeddings, position_embeddings)
    emb = emb_flat.reshape(BATCH, SEQ, D_MODEL)
    wo = wo_flat.reshape(BATCH, SEQ, D_MODEL)
    return (emb, None, wo)
